# baseline (speedup 1.0000x reference)
import jax
import jax.numpy as jnp
from jax import lax
from jax.experimental import pallas as pl
from jax.experimental.pallas import tpu as pltpu

BSZ, SEQ, ENC = 8, 2048, 512
HID = 512
T = 16


def _fused_body(x_ref, ends_ref, prev_ref, invc_ref, wih_ref, whh_ref,
                bias_ref, wfc_ref, bfc_ref, out_ref, st_ref, g_ref):
    bi = pl.program_id(0)
    x = x_ref[0]
    ends = ends_ref[0]
    prev = prev_ref[0]
    invc = invc_ref[0]
    pos = lax.broadcasted_iota(jnp.int32, (SEQ, T), 0)
    m = jnp.where((pos > prev) & (pos <= ends), invc, 0.0)
    contrib = lax.dot_general(m, x, (((0,), (0,)), ((), ())),
                              preferred_element_type=jnp.float32)
    st_ref[pl.ds(bi * T, T), :] = contrib
    g_ref[pl.ds(bi * T, T), :] = lax.dot_general(
        contrib, wih_ref[...], (((1,), (1,)), ((), ())),
        preferred_element_type=jnp.float32)

    @pl.when(bi == BSZ - 1)
    def _tail():
        sf = st_ref[...]
        states = sf.reshape(BSZ, T, ENC)
        ws = wfc_ref[0:1, HID:HID + ENC]
        bs3 = jnp.concatenate(
            [lax.dot_general(ws, states[b], (((1,), (1,)), ((), ())),
                             preferred_element_type=jnp.float32
                             ).reshape(1, 1, T)
             for b in range(BSZ)], axis=0)
        bias = bias_ref[...]
        G = g_ref[...] + bias
        G3 = G.reshape(BSZ, T, 4 * HID)

        h = jnp.zeros((BSZ * T, HID), jnp.float32)
        c = jnp.zeros((BSZ * T, HID), jnp.float32)
        for t in range(3):
            if t == 0:
                xg = G
            else:
                xg = jnp.concatenate([G3[:, t:, :], G3[:, :t, :]],
                                     axis=1).reshape(BSZ * T, 4 * HID)
            gates = xg + lax.dot_general(
                h.astype(jnp.bfloat16), whh_ref[...], (((1,), (1,)), ((), ())),
                preferred_element_type=jnp.float32)
            i_g = jax.nn.sigmoid(gates[:, 0:HID])
            f_g = jax.nn.sigmoid(gates[:, HID:2 * HID])
            g_g = jnp.tanh(gates[:, 2 * HID:3 * HID])
            o_g = jax.nn.sigmoid(gates[:, 3 * HID:4 * HID])
            c = f_g * c + i_g * g_g
            h = o_g * jnp.tanh(c)

        wh = wfc_ref[0:1, 0:HID]
        a = jnp.sum(h * wh, axis=1, keepdims=True) + bfc_ref[0, 0]
        a3 = a.reshape(BSZ, T, 1)
        logits = a3 + bs3
        j_idx = lax.broadcasted_iota(jnp.int32, (BSZ, T, T), 1)
        k_idx = lax.broadcasted_iota(jnp.int32, (BSZ, T, T), 2)
        valid = k_idx >= (j_idx + 3)
        lm = jnp.where(valid, logits, -1e30)
        mx = jnp.max(lm, axis=2, keepdims=True)
        logz = mx + jnp.log(jnp.sum(jnp.exp(lm - mx), axis=2, keepdims=True))
        tgt = jnp.sum(jnp.where(k_idx == j_idx + 3, logits, 0.0),
                      axis=2, keepdims=True)
        val = logz - tgt
        jmask = lax.broadcasted_iota(jnp.int32, (BSZ, T, 1), 1) < (T - 3)
        out_ref[0, 0] = jnp.sum(jnp.where(jmask, val, 0.0)) / (BSZ * (T - 3))


def kernel(encoder_output, his_turn_end_ids, W_ih, W_hh, b_ih, b_hh,
           W_fc, b_fc):
    ends = his_turn_end_ids.astype(jnp.int32)
    prev = jnp.concatenate(
        [jnp.full((BSZ, 1), -1, jnp.int32), ends[:, :-1]], axis=1)
    invc = 1.0 / (ends - prev).astype(jnp.float32)
    ends3 = ends.reshape(BSZ, 1, T)
    prev3 = prev.reshape(BSZ, 1, T)
    invc3 = invc.reshape(BSZ, 1, T)
    bias = (b_ih + b_hh).reshape(1, 4 * HID)
    bfc = b_fc.reshape(1, 1)
    W_hh_bf = W_hh.astype(jnp.bfloat16)

    loss2d = pl.pallas_call(
        _fused_body,
        grid=(BSZ,),
        in_specs=[
            pl.BlockSpec((1, SEQ, ENC), lambda b: (b, 0, 0)),
            pl.BlockSpec((1, 1, T), lambda b: (b, 0, 0)),
            pl.BlockSpec((1, 1, T), lambda b: (b, 0, 0)),
            pl.BlockSpec((1, 1, T), lambda b: (b, 0, 0)),
            pl.BlockSpec((4 * HID, ENC), lambda b: (0, 0)),
            pl.BlockSpec((4 * HID, HID), lambda b: (0, 0)),
            pl.BlockSpec((1, 4 * HID), lambda b: (0, 0)),
            pl.BlockSpec((1, HID + ENC), lambda b: (0, 0)),
            pl.BlockSpec(memory_space=pltpu.SMEM),
        ],
        out_specs=pl.BlockSpec(memory_space=pltpu.SMEM),
        out_shape=jax.ShapeDtypeStruct((1, 1), jnp.float32),
        scratch_shapes=[pltpu.VMEM((BSZ * T, ENC), jnp.float32),
                        pltpu.VMEM((BSZ * T, 4 * HID), jnp.float32)],
    )(encoder_output, ends3, prev3, invc3, W_ih, W_hh_bf, bias, W_fc, bfc)
    return loss2d[0, 0]


# speedup vs baseline: 1.0610x; 1.0610x over previous
import jax
import jax.numpy as jnp
from jax import lax
from jax.experimental import pallas as pl
from jax.experimental.pallas import tpu as pltpu

BSZ, SEQ, ENC = 8, 2048, 512
HID = 512
T = 16


def _fused_body(x_ref, ends_ref, prev_ref, invc_ref, wih_ref, whh_ref,
                bias_ref, wfc_ref, bfc_ref, out_ref, st_ref):
    bi = pl.program_id(0)
    x = x_ref[0]
    ends = ends_ref[0]
    prev = prev_ref[0]
    invc = invc_ref[0]
    pos = lax.broadcasted_iota(jnp.int32, (SEQ, T), 0)
    m = jnp.where((pos > prev) & (pos <= ends), invc, 0.0)
    st_ref[pl.ds(bi * T, T), :] = lax.dot_general(
        m, x, (((0,), (0,)), ((), ())),
        preferred_element_type=jnp.float32)

    @pl.when(bi == BSZ - 1)
    def _tail():
        sf = st_ref[...]
        states = sf.reshape(BSZ, T, ENC)
        ws = wfc_ref[0:1, HID:HID + ENC]
        bs3 = jnp.concatenate(
            [lax.dot_general(ws, states[b], (((1,), (1,)), ((), ())),
                             preferred_element_type=jnp.float32
                             ).reshape(1, 1, T)
             for b in range(BSZ)], axis=0)
        bias = bias_ref[...]
        G = lax.dot_general(sf, wih_ref[...], (((1,), (1,)), ((), ())),
                            preferred_element_type=jnp.float32) + bias
        G3 = G.reshape(BSZ, T, 4 * HID)

        h = jnp.zeros((BSZ * T, HID), jnp.float32)
        c = jnp.zeros((BSZ * T, HID), jnp.float32)
        for t in range(3):
            if t == 0:
                xg = G
            else:
                xg = jnp.concatenate([G3[:, t:, :], G3[:, :t, :]],
                                     axis=1).reshape(BSZ * T, 4 * HID)
            gates = xg + lax.dot_general(
                h.astype(jnp.bfloat16), whh_ref[...], (((1,), (1,)), ((), ())),
                preferred_element_type=jnp.float32)
            i_g = jax.nn.sigmoid(gates[:, 0:HID])
            f_g = jax.nn.sigmoid(gates[:, HID:2 * HID])
            g_g = jnp.tanh(gates[:, 2 * HID:3 * HID])
            o_g = jax.nn.sigmoid(gates[:, 3 * HID:4 * HID])
            c = f_g * c + i_g * g_g
            h = o_g * jnp.tanh(c)

        wh = wfc_ref[0:1, 0:HID]
        a = jnp.sum(h * wh, axis=1, keepdims=True) + bfc_ref[0, 0]
        a3 = a.reshape(BSZ, T, 1)
        logits = a3 + bs3
        j_idx = lax.broadcasted_iota(jnp.int32, (BSZ, T, T), 1)
        k_idx = lax.broadcasted_iota(jnp.int32, (BSZ, T, T), 2)
        valid = k_idx >= (j_idx + 3)
        lm = jnp.where(valid, logits, -1e30)
        mx = jnp.max(lm, axis=2, keepdims=True)
        logz = mx + jnp.log(jnp.sum(jnp.exp(lm - mx), axis=2, keepdims=True))
        tgt = jnp.sum(jnp.where(k_idx == j_idx + 3, logits, 0.0),
                      axis=2, keepdims=True)
        val = logz - tgt
        jmask = lax.broadcasted_iota(jnp.int32, (BSZ, T, 1), 1) < (T - 3)
        out_ref[0, 0] = jnp.sum(jnp.where(jmask, val, 0.0)) / (BSZ * (T - 3))


def kernel(encoder_output, his_turn_end_ids, W_ih, W_hh, b_ih, b_hh,
           W_fc, b_fc):
    ends = his_turn_end_ids.astype(jnp.int32)
    prev = jnp.concatenate(
        [jnp.full((BSZ, 1), -1, jnp.int32), ends[:, :-1]], axis=1)
    invc = 1.0 / (ends - prev).astype(jnp.float32)
    ends3 = ends.reshape(BSZ, 1, T)
    prev3 = prev.reshape(BSZ, 1, T)
    invc3 = invc.reshape(BSZ, 1, T)
    bias = (b_ih + b_hh).reshape(1, 4 * HID)
    bfc = b_fc.reshape(1, 1)
    W_hh_bf = W_hh.astype(jnp.bfloat16)

    loss2d = pl.pallas_call(
        _fused_body,
        grid=(BSZ,),
        in_specs=[
            pl.BlockSpec((1, SEQ, ENC), lambda b: (b, 0, 0)),
            pl.BlockSpec((1, 1, T), lambda b: (b, 0, 0)),
            pl.BlockSpec((1, 1, T), lambda b: (b, 0, 0)),
            pl.BlockSpec((1, 1, T), lambda b: (b, 0, 0)),
            pl.BlockSpec((4 * HID, ENC), lambda b: (0, 0)),
            pl.BlockSpec((4 * HID, HID), lambda b: (0, 0)),
            pl.BlockSpec((1, 4 * HID), lambda b: (0, 0)),
            pl.BlockSpec((1, HID + ENC), lambda b: (0, 0)),
            pl.BlockSpec(memory_space=pltpu.SMEM),
        ],
        out_specs=pl.BlockSpec(memory_space=pltpu.SMEM),
        out_shape=jax.ShapeDtypeStruct((1, 1), jnp.float32),
        scratch_shapes=[pltpu.VMEM((BSZ * T, ENC), jnp.float32)],
    )(encoder_output, ends3, prev3, invc3, W_ih, W_hh_bf, bias, W_fc, bfc)
    return loss2d[0, 0]


# speedup vs baseline: 1.1984x; 1.1295x over previous
import jax
import jax.numpy as jnp
from jax import lax
from jax.experimental import pallas as pl
from jax.experimental.pallas import tpu as pltpu

BSZ, SEQ, ENC = 8, 2048, 512
HID = 512
T = 16


def _fused_body(x_ref, ends_ref, prev_ref, invc_ref, wih_ref, whh_ref,
                bias_ref, wfc_ref, bfc_ref, out_ref, st_ref):
    bi = pl.program_id(0)
    x = x_ref[0]
    ends = ends_ref[0]
    prev = prev_ref[0]
    invc = invc_ref[0]
    pos = lax.broadcasted_iota(jnp.int32, (SEQ, T), 0)
    m = jnp.where((pos > prev) & (pos <= ends), invc, 0.0)
    st_ref[pl.ds(bi * T, T), :] = lax.dot_general(
        m, x, (((0,), (0,)), ((), ())),
        preferred_element_type=jnp.float32)

    @pl.when(bi == BSZ - 1)
    def _tail():
        sf = st_ref[...]
        states = sf.reshape(BSZ, T, ENC)
        ws = wfc_ref[0:1, HID:HID + ENC]
        bs3 = jnp.concatenate(
            [lax.dot_general(ws, states[b], (((1,), (1,)), ((), ())),
                             preferred_element_type=jnp.float32
                             ).reshape(1, 1, T)
             for b in range(BSZ)], axis=0)
        bias = bias_ref[...]
        G = lax.dot_general(sf, wih_ref[...], (((1,), (1,)), ((), ())),
                            preferred_element_type=jnp.float32) + bias
        G3 = G.reshape(BSZ, T, 4 * HID)

        whh_bf = whh_ref[...].astype(jnp.bfloat16)
        h = jnp.zeros((BSZ * T, HID), jnp.float32)
        c = jnp.zeros((BSZ * T, HID), jnp.float32)
        for t in range(3):
            if t == 0:
                xg = G
            else:
                xg = jnp.concatenate([G3[:, t:, :], G3[:, :t, :]],
                                     axis=1).reshape(BSZ * T, 4 * HID)
            gates = xg + lax.dot_general(
                h.astype(jnp.bfloat16), whh_bf, (((1,), (1,)), ((), ())),
                preferred_element_type=jnp.float32)
            i_g = jax.nn.sigmoid(gates[:, 0:HID])
            f_g = jax.nn.sigmoid(gates[:, HID:2 * HID])
            g_g = jnp.tanh(gates[:, 2 * HID:3 * HID])
            o_g = jax.nn.sigmoid(gates[:, 3 * HID:4 * HID])
            c = f_g * c + i_g * g_g
            h = o_g * jnp.tanh(c)

        wh = wfc_ref[0:1, 0:HID]
        a = jnp.sum(h * wh, axis=1, keepdims=True) + bfc_ref[0, 0]
        a3 = a.reshape(BSZ, T, 1)
        logits = a3 + bs3
        j_idx = lax.broadcasted_iota(jnp.int32, (BSZ, T, T), 1)
        k_idx = lax.broadcasted_iota(jnp.int32, (BSZ, T, T), 2)
        valid = k_idx >= (j_idx + 3)
        lm = jnp.where(valid, logits, -1e30)
        mx = jnp.max(lm, axis=2, keepdims=True)
        logz = mx + jnp.log(jnp.sum(jnp.exp(lm - mx), axis=2, keepdims=True))
        tgt = jnp.sum(jnp.where(k_idx == j_idx + 3, logits, 0.0),
                      axis=2, keepdims=True)
        val = logz - tgt
        jmask = lax.broadcasted_iota(jnp.int32, (BSZ, T, 1), 1) < (T - 3)
        out_ref[0, 0] = jnp.sum(jnp.where(jmask, val, 0.0)) / (BSZ * (T - 3))


def kernel(encoder_output, his_turn_end_ids, W_ih, W_hh, b_ih, b_hh,
           W_fc, b_fc):
    ends = his_turn_end_ids.astype(jnp.int32)
    prev = jnp.concatenate(
        [jnp.full((BSZ, 1), -1, jnp.int32), ends[:, :-1]], axis=1)
    invc = 1.0 / (ends - prev).astype(jnp.float32)
    ends3 = ends.reshape(BSZ, 1, T)
    prev3 = prev.reshape(BSZ, 1, T)
    invc3 = invc.reshape(BSZ, 1, T)
    bias = (b_ih + b_hh).reshape(1, 4 * HID)
    bfc = b_fc.reshape(1, 1)

    loss2d = pl.pallas_call(
        _fused_body,
        grid=(BSZ,),
        in_specs=[
            pl.BlockSpec((1, SEQ, ENC), lambda b: (b, 0, 0)),
            pl.BlockSpec((1, 1, T), lambda b: (b, 0, 0)),
            pl.BlockSpec((1, 1, T), lambda b: (b, 0, 0)),
            pl.BlockSpec((1, 1, T), lambda b: (b, 0, 0)),
            pl.BlockSpec((4 * HID, ENC), lambda b: (0, 0)),
            pl.BlockSpec((4 * HID, HID), lambda b: (0, 0)),
            pl.BlockSpec((1, 4 * HID), lambda b: (0, 0)),
            pl.BlockSpec((1, HID + ENC), lambda b: (0, 0)),
            pl.BlockSpec(memory_space=pltpu.SMEM),
        ],
        out_specs=pl.BlockSpec(memory_space=pltpu.SMEM),
        out_shape=jax.ShapeDtypeStruct((1, 1), jnp.float32),
        scratch_shapes=[pltpu.VMEM((BSZ * T, ENC), jnp.float32)],
    )(encoder_output, ends3, prev3, invc3, W_ih, W_hh, bias, W_fc, bfc)
    return loss2d[0, 0]
